# Initial kernel scaffold; baseline (speedup 1.0000x reference)
import jax
import jax.numpy as jnp
from jax.experimental import pallas as pl


def kernel(idx, weights):
    raise NotImplementedError("write your pallas kernel here")


# speedup vs baseline: 11.3947x; 11.3947x over previous
import functools

import jax
import jax.numpy as jnp
from jax import lax
from jax.experimental import pallas as pl
from jax.experimental.pallas import tpu as pltpu
from jax.experimental.pallas import tpu_sc as plsc

NUM_EMBEDDINGS = 1000000
EMBEDDING_DIM = 32
BATCH = 16384
N_FIELDS = 26

_INFO = plsc.get_sparse_core_info()
NC = _INFO.num_cores
NS = _INFO.num_subcores
NW = NC * NS

N = BATCH * N_FIELDS
PER_W = N // NW
CHUNK = 128
NCH = PER_W // CHUNK

_mesh = plsc.VectorSubcoreMesh(core_axis_name="c", subcore_axis_name="s")


@functools.partial(
    pl.kernel,
    out_type=jax.ShapeDtypeStruct((N, EMBEDDING_DIM), jnp.float32),
    mesh=_mesh,
    scratch_types=[
        pltpu.VMEM((NCH, CHUNK), jnp.int32),
        pltpu.VMEM((CHUNK, EMBEDDING_DIM), jnp.float32),
        pltpu.SemaphoreType.DMA,
    ],
    compiler_params=pltpu.CompilerParams(use_tc_tiling_on_sc=False),
)
def _gather_kernel(idx_hbm, tbl_hbm, out_hbm, idx_v, rows_v, gsem):
    wid = lax.axis_index("s") * NC + lax.axis_index("c")
    pltpu.sync_copy(idx_hbm.at[wid], idx_v)
    row_base = wid * PER_W

    def step(j, carry):
        pltpu.async_copy(tbl_hbm.at[idx_v.at[j]], rows_v, gsem).wait()
        pltpu.sync_copy(rows_v, out_hbm.at[pl.ds(row_base + j * CHUNK, CHUNK)])
        return carry

    lax.fori_loop(0, NCH, step, 0)


def kernel(idx, weights):
    idx_w = idx.reshape(NW, NCH, CHUNK).astype(jnp.int32)
    out = _gather_kernel(idx_w, weights)
    return out.reshape(BATCH, N_FIELDS, EMBEDDING_DIM).astype(jnp.bfloat16)
